# baseline (speedup 1.0000x reference)
import functools

import jax
import jax.numpy as jnp
from jax import lax
from jax.experimental import pallas as pl
from jax.experimental.pallas import tpu as pltpu
from jax.experimental.pallas import tpu_sc as plsc

N = 10000
E = 320000
D = 128
K = 5
ITERS = 25

NPAD = 10240
DUMP = NPAD - 1
NC, NS, L = 2, 16, 16
NW = NC * NS
CH = 128
NCHUNK = 79
EPT = NCHUNK * CH
EP = EPT * NW
NODES_PER_TILE = NPAD // NS

NCH4 = 160
GRP = 16
NGRP = NCH4 // GRP
EPP = NS * NCH4 * CH


def _sc_mesh():
    return plsc.VectorSubcoreMesh(core_axis_name="c", subcore_axis_name="s")


_SC1_MESH = plsc.VectorSubcoreMesh(core_axis_name="c", subcore_axis_name="s",
                                   num_cores=1)
_SC_PARAMS = pltpu.CompilerParams(needs_layout_passes=False)


@functools.partial(
    pl.kernel,
    out_type=jax.ShapeDtypeStruct((NC, NPAD), jnp.float32),
    mesh=_sc_mesh(),
    scratch_types=[
        pltpu.VMEM((EPT,), jnp.int32),
        pltpu.VMEM((NPAD,), jnp.float32),
        pltpu.VMEM((NODES_PER_TILE,), jnp.float32),
        pltpu.VMEM((NODES_PER_TILE,), jnp.float32),
        pltpu.VMEM_SHARED((NS, NPAD), jnp.float32),
    ],
    compiler_params=_SC_PARAMS,
)
def _deg_sc(dst_hbm, out_hbm, dst_v, hist_v, acc_v, tmp_v, stage_sh):
    cid = lax.axis_index("c")
    sid = lax.axis_index("s")
    wid = cid * NS + sid
    zeros = jnp.zeros((L,), jnp.float32)
    ones = jnp.ones((L,), jnp.float32)

    def zero_body(i, _):
        hist_v[pl.ds(i * L, L)] = zeros
        return 0
    lax.fori_loop(0, NPAD // L, zero_body, 0)

    pltpu.sync_copy(dst_hbm.at[pl.ds(wid * EPT, EPT)], dst_v)

    def scat_body(i, _):
        idx = dst_v[pl.ds(i * L, L)]
        plsc.addupdate_scatter(hist_v, [idx], ones)
        return 0
    lax.fori_loop(0, EPT // L, scat_body, 0)

    pltpu.sync_copy(hist_v, stage_sh.at[sid])
    plsc.subcore_barrier()

    base = sid * NODES_PER_TILE
    def merge_zero(i, _):
        acc_v[pl.ds(i * L, L)] = zeros
        return 0
    lax.fori_loop(0, NODES_PER_TILE // L, merge_zero, 0)
    for t in range(NS):
        pltpu.sync_copy(stage_sh.at[t, pl.ds(base, NODES_PER_TILE)], tmp_v)
        def add_body(i, _):
            acc_v[pl.ds(i * L, L)] = acc_v[pl.ds(i * L, L)] + tmp_v[pl.ds(i * L, L)]
            return 0
        lax.fori_loop(0, NODES_PER_TILE // L, add_body, 0)
    pltpu.sync_copy(acc_v, out_hbm.at[cid, pl.ds(base, NODES_PER_TILE)])


def _degfix_body(p0_ref, p1_ref, x_ref, norm_ref, inv_ref, normx_ref):
    deg = p0_ref[...] + p1_ref[...] + 1.0
    nrm = jax.lax.rsqrt(deg)
    norm_ref[...] = nrm
    inv_ref[...] = 1.0 / deg
    normx_ref[...] = x_ref[...] * nrm


@jax.jit
def _deg_stage(dst, x_pad):
    partials = _deg_sc(dst)
    norm, inv, normx = pl.pallas_call(
        _degfix_body,
        out_shape=(
            jax.ShapeDtypeStruct((NPAD, 1), jnp.float32),
            jax.ShapeDtypeStruct((NPAD, 1), jnp.float32),
            jax.ShapeDtypeStruct((NPAD, D), jnp.float32),
        ),
    )(partials[0][:, None], partials[1][:, None], x_pad)
    return norm.reshape(NPAD), inv.reshape(NPAD), normx


@functools.partial(
    pl.kernel,
    out_type=jax.ShapeDtypeStruct((NC, NPAD, D), jnp.float32),
    mesh=_sc_mesh(),
    scratch_types=[
        pltpu.VMEM((NCHUNK, CH), jnp.int32),
        pltpu.VMEM((NCHUNK, CH), jnp.int32),
        pltpu.VMEM((CH, D), jnp.float32),
        pltpu.VMEM_SHARED((NPAD, D), jnp.float32),
        pltpu.SemaphoreType.DMA,
    ],
    compiler_params=_SC_PARAMS,
)
def _agg_sc(src_hbm, dst_hbm, nx_hbm, out_hbm,
            src_v, dst_v, rows_a, acc_sh, sem_a):
    cid = lax.axis_index("c")
    sid = lax.axis_index("s")
    wid = cid * NS + sid
    zeros = jnp.zeros((L,), jnp.float32)

    pltpu.sync_copy(src_hbm.at[wid], src_v)
    pltpu.sync_copy(dst_hbm.at[wid], dst_v)

    def z_rows(r, _):
        for v in range(D // L):
            rows_a[r, pl.ds(v * L, L)] = zeros
        return 0
    lax.fori_loop(0, CH, z_rows, 0)
    base = sid * NODES_PER_TILE
    for b in range(NODES_PER_TILE // CH):
        pltpu.sync_copy(rows_a, acc_sh.at[pl.ds(base + b * CH, CH)])
    plsc.subcore_barrier()

    def chunk_body(j, _):
        pltpu.async_copy(nx_hbm.at[src_v.at[j]], rows_a, sem_a).wait()
        pltpu.sync_copy(rows_a, acc_sh.at[dst_v.at[j]], add=True)
        return 0
    lax.fori_loop(0, NCHUNK, chunk_body, 0)

    plsc.subcore_barrier()
    pltpu.sync_copy(acc_sh.at[pl.ds(base, NODES_PER_TILE)],
                    out_hbm.at[cid, pl.ds(base, NODES_PER_TILE)])


ROWS_BLK = 1280


def _enc_body(p0_ref, p1_ref, x_ref, norm_ref, inv_ref, w_ref, b_ref,
              abs_ref, kn_ref, qn_ref):
    i = pl.program_id(0)
    xagg = (norm_ref[...] * (p0_ref[...] + p1_ref[...])
            + x_ref[...] * inv_ref[...])
    y = jax.lax.dot_general(xagg, w_ref[...], (((1,), (0,)), ((), ())),
                            preferred_element_type=jnp.float32) + b_ref[...]
    ab, kk, qq = y[:, :D], y[:, D:2 * D], y[:, 2 * D:]
    row = jax.lax.broadcasted_iota(jnp.int32, (ROWS_BLK, 1), 0) + i * ROWS_BLK
    valid = row < N
    nk = jnp.sqrt(jnp.sum(kk * kk, axis=1, keepdims=True))
    nq = jnp.sqrt(jnp.sum(qq * qq, axis=1, keepdims=True))
    kn = jnp.where(valid, kk / jnp.maximum(nk, 1e-8), 0.0)
    qn = jnp.where(valid, qq / jnp.maximum(nq, 1e-8), 0.0)
    abs_ref[...] = ab
    kn_ref[...] = kn
    qn_ref[...] = qn


@jax.jit
def _encode(p, x_pad, norm_full, inv_full, wcat, bcat):
    grid = NPAD // ROWS_BLK
    blk = lambda i: (i, 0)
    return pl.pallas_call(
        _enc_body,
        grid=(grid,),
        in_specs=[
            pl.BlockSpec((ROWS_BLK, D), blk),
            pl.BlockSpec((ROWS_BLK, D), blk),
            pl.BlockSpec((ROWS_BLK, D), blk),
            pl.BlockSpec((ROWS_BLK, 1), blk),
            pl.BlockSpec((ROWS_BLK, 1), blk),
            pl.BlockSpec((D, 3 * D), lambda i: (0, 0)),
            pl.BlockSpec((1, 3 * D), lambda i: (0, 0)),
        ],
        out_specs=[
            pl.BlockSpec((ROWS_BLK, D), blk),
            pl.BlockSpec((ROWS_BLK, D), blk),
            pl.BlockSpec((ROWS_BLK, D), blk),
        ],
        out_shape=[
            jax.ShapeDtypeStruct((NPAD, D), jnp.float32),
            jax.ShapeDtypeStruct((NPAD, D), jnp.float32),
            jax.ShapeDtypeStruct((NPAD, D), jnp.float32),
        ],
    )(p[0], p[1], x_pad, norm_full[:, None], inv_full[:, None], wcat,
      bcat[None, :])


@functools.partial(
    pl.kernel,
    out_type=(
        jax.ShapeDtypeStruct((NW, NCHUNK, CH), jnp.float32),
        jax.ShapeDtypeStruct((NC, NPAD), jnp.float32),
    ),
    mesh=_sc_mesh(),
    scratch_types=[
        pltpu.VMEM((NCHUNK, CH), jnp.int32),
        pltpu.VMEM((NCHUNK, CH), jnp.int32),
        pltpu.VMEM((NCHUNK, CH), jnp.float32),
        pltpu.VMEM((NPAD,), jnp.float32),
        pltpu.VMEM((NODES_PER_TILE,), jnp.float32),
        pltpu.VMEM((CH, D), jnp.float32),
        pltpu.VMEM((CH, D), jnp.float32),
        pltpu.VMEM_SHARED((NS, NPAD), jnp.float32),
        pltpu.SemaphoreType.DMA,
        pltpu.SemaphoreType.DMA,
    ],
    compiler_params=_SC_PARAMS,
)
def _cos_sc(src_hbm, dst_hbm, kn_hbm, qn_hbm, cos_hbm, m_hbm,
            src_v, dst_v, cos_v, m_v, tmp_v, rows_k, rows_q, stage_sh,
            sem_k, sem_q):
    cid = lax.axis_index("c")
    sid = lax.axis_index("s")
    wid = cid * NS + sid
    neg_inf = jnp.full((L,), -jnp.inf, jnp.float32)

    pltpu.sync_copy(src_hbm.at[wid], src_v)
    pltpu.sync_copy(dst_hbm.at[wid], dst_v)

    def minit(i, _):
        m_v[pl.ds(i * L, L)] = neg_inf
        return 0
    lax.fori_loop(0, NPAD // L, minit, 0)

    lane = lax.iota(jnp.int32, L)

    def chunk_body(j, _):
        ck = pltpu.async_copy(kn_hbm.at[src_v.at[j]], rows_k, sem_k)
        cq = pltpu.async_copy(qn_hbm.at[dst_v.at[j]], rows_q, sem_q)
        ck.wait()
        cq.wait()
        def group_body(t, _):
            ri = t * L + lane
            def col_body(v4, acc):
                for u in range(4):
                    vb = jnp.full((L,), v4 * 4 + u, jnp.int32)
                    ka = plsc.load_gather(rows_k, [ri, vb])
                    qa = plsc.load_gather(rows_q, [ri, vb])
                    acc = acc + ka * qa
                return acc
            dots = lax.fori_loop(0, D // 4, col_body,
                                 jnp.zeros((L,), jnp.float32))
            cos_v[j, pl.ds(t * L, L)] = dots
            d16 = dst_v[j, pl.ds(t * L, L)]
            rank, _last = plsc.scan_count(d16)
            def round_body(k, _):
                mg = plsc.load_gather(m_v, [d16])
                mx = jnp.maximum(mg, dots)
                plsc.store_scatter(m_v, [d16], mx, mask=rank == k)
                return 0
            lax.fori_loop(0, L + 1, round_body, 0)
            return 0
        lax.fori_loop(0, CH // L, group_body, 0)
        return 0
    lax.fori_loop(0, NCHUNK, chunk_body, 0)

    pltpu.sync_copy(cos_v, cos_hbm.at[wid])
    pltpu.sync_copy(m_v, stage_sh.at[sid])
    plsc.subcore_barrier()

    base = sid * NODES_PER_TILE
    def mz(i, _):
        tmp_v[pl.ds(i * L, L)] = neg_inf
        return 0
    lax.fori_loop(0, NODES_PER_TILE // L, mz, 0)
    for t in range(NS):
        pltpu.sync_copy(stage_sh.at[t, pl.ds(base, NODES_PER_TILE)],
                        m_v.at[pl.ds(0, NODES_PER_TILE)])
        def mmax(i, _):
            sl = pl.ds(i * L, L)
            tmp_v[sl] = jnp.maximum(tmp_v[sl], m_v[sl])
            return 0
        lax.fori_loop(0, NODES_PER_TILE // L, mmax, 0)
    pltpu.sync_copy(tmp_v, m_hbm.at[cid, pl.ds(base, NODES_PER_TILE)])


@functools.partial(
    pl.kernel,
    out_type=jax.ShapeDtypeStruct((NW, NCHUNK, CH), jnp.float32),
    mesh=_sc_mesh(),
    scratch_types=[
        pltpu.VMEM((NCHUNK, CH), jnp.int32),
        pltpu.VMEM((NCHUNK, CH), jnp.float32),
        pltpu.VMEM((NPAD,), jnp.float32),
        pltpu.VMEM((NPAD,), jnp.float32),
    ],
    compiler_params=_SC_PARAMS,
)
def _w_sc(dst_hbm, cos_hbm, m_hbm, w_hbm, dst_v, w_v, m_v, tmp_v):
    cid = lax.axis_index("c")
    sid = lax.axis_index("s")
    wid = cid * NS + sid

    pltpu.sync_copy(dst_hbm.at[wid], dst_v)
    pltpu.sync_copy(cos_hbm.at[wid], w_v)
    pltpu.sync_copy(m_hbm.at[0], m_v)
    pltpu.sync_copy(m_hbm.at[1], tmp_v)

    def mmerge(i, _):
        sl = pl.ds(i * L, L)
        m_v[sl] = jnp.maximum(m_v[sl], tmp_v[sl])
        return 0
    lax.fori_loop(0, NPAD // L, mmerge, 0)

    def wchunk(j, _):
        def winner(t, _):
            sl = pl.ds(t * L, L)
            mg = plsc.load_gather(m_v, [dst_v[j, sl]])
            w_v[j, sl] = jnp.exp(w_v[j, sl] - mg)
            return 0
        return lax.fori_loop(0, CH // L, winner, 0)
    lax.fori_loop(0, NCHUNK, wchunk, 0)
    pltpu.sync_copy(w_v, w_hbm.at[wid])


@functools.partial(
    pl.kernel,
    out_type=jax.ShapeDtypeStruct((NPAD, D), jnp.float32),
    mesh=_SC1_MESH,
    scratch_types=[
        pltpu.VMEM((GRP, CH), jnp.int32),
        pltpu.VMEM((GRP, CH), jnp.int32),
        pltpu.VMEM((GRP, CH), jnp.int32),
        pltpu.VMEM((GRP, CH), jnp.int32),
        pltpu.VMEM((GRP, CH), jnp.float32),
        pltpu.VMEM((GRP, CH), jnp.float32),
        pltpu.VMEM((CH, D), jnp.float32),
        pltpu.VMEM((CH, D), jnp.float32),
        pltpu.VMEM_SHARED((NPAD, D), jnp.float32),
        pltpu.SemaphoreType.DMA,
        pltpu.SemaphoreType.DMA,
        pltpu.SemaphoreType.DMA,
        pltpu.SemaphoreType.DMA,
        pltpu.SemaphoreType.DMA,
        pltpu.SemaphoreType.DMA,
    ],
    compiler_params=_SC_PARAMS,
)
def _prop_sc(src_hbm, dst_hbm, w_hbm, h0_hbm, h_hbm,
             src0_v, src1_v, dst0_v, dst1_v, w0_v, w1_v, rows_a, rows_b,
             acc_sh, sem_a, sem_b, sem_i0, sem_i1, sem_wa, sem_wb):
    sid = lax.axis_index("s")
    base = sid * NODES_PER_TILE
    zeros = jnp.zeros((L,), jnp.float32)
    sbuf = (src0_v, src1_v)
    dbuf = (dst0_v, dst1_v)
    wbuf = (w0_v, w1_v)
    ibuf_sem = (sem_i0, sem_i1)
    rbuf = (rows_a, rows_b)
    gsem = (sem_a, sem_b)
    wsem = (sem_wa, sem_wb)

    def z_rows(r, _):
        for v in range(D // L):
            rows_a[r, pl.ds(v * L, L)] = zeros
        return 0
    lax.fori_loop(0, CH, z_rows, 0)
    for b in range(NODES_PER_TILE // CH):
        pltpu.sync_copy(rows_a, acc_sh.at[pl.ds(base + b * CH, CH)])
    for b in range(NODES_PER_TILE // CH):
        sl = pl.ds(base + b * CH, CH)
        pltpu.sync_copy(h0_hbm.at[sl], rows_a)
        pltpu.sync_copy(rows_a, h_hbm.at[sl])
    plsc.subcore_barrier()

    def issue_grp(g, b):
        gs = pl.ds(g * GRP, GRP)
        pltpu.async_copy(src_hbm.at[sid, gs], sbuf[b], ibuf_sem[b])
        pltpu.async_copy(dst_hbm.at[sid, gs], dbuf[b], ibuf_sem[b])
        pltpu.async_copy(w_hbm.at[sid, gs], wbuf[b], ibuf_sem[b])

    def wait_grp(g, b):
        gs = pl.ds(g * GRP, GRP)
        pltpu.make_async_copy(src_hbm.at[sid, gs], sbuf[b], ibuf_sem[b]).wait()
        pltpu.make_async_copy(dst_hbm.at[sid, gs], dbuf[b], ibuf_sem[b]).wait()
        pltpu.make_async_copy(w_hbm.at[sid, gs], wbuf[b], ibuf_sem[b]).wait()

    def scale(wv, rows, jj):
        jb = jnp.full((L,), jj, jnp.int32)
        def scale4(ei, _):
            for u in range(4):
                e = ei * 4 + u
                wb = plsc.load_gather(wv, [jb, jnp.full((L,), e, jnp.int32)])
                for v in range(D // L):
                    sl = pl.ds(v * L, L)
                    rows[e, sl] = rows[e, sl] * wb
            return 0
        lax.fori_loop(0, CH // 4, scale4, 0)

    def iter_body(it, _):
        issue_grp(0, 0)
        for g in range(NGRP):
            b = g % 2
            sv, dv, wv = sbuf[b], dbuf[b], wbuf[b]
            wait_grp(g, b)
            if g + 1 < NGRP:
                issue_grp(g + 1, 1 - b)
            pltpu.async_copy(h_hbm.at[sv.at[0]], rows_a, sem_a)
            def pair_body(p, _):
                j0 = 2 * p
                @pl.when(j0 >= 1)
                def _():
                    pltpu.make_async_copy(rows_b, acc_sh.at[dv.at[j0 - 1]],
                                          sem_wb).wait()
                pltpu.async_copy(h_hbm.at[sv.at[j0 + 1]], rows_b, sem_b)
                pltpu.make_async_copy(h_hbm.at[sv.at[j0]], rows_a,
                                      sem_a).wait()
                scale(wv, rows_a, j0)
                pltpu.async_copy(rows_a, acc_sh.at[dv.at[j0]], sem_wa,
                                 add=True)
                pltpu.make_async_copy(h_hbm.at[sv.at[j0 + 1]], rows_b,
                                      sem_b).wait()
                scale(wv, rows_b, j0 + 1)
                pltpu.async_copy(rows_b, acc_sh.at[dv.at[j0 + 1]], sem_wb,
                                 add=True)
                pltpu.make_async_copy(rows_a, acc_sh.at[dv.at[j0]],
                                      sem_wa).wait()
                @pl.when(p < GRP // 2 - 1)
                def _():
                    pltpu.async_copy(h_hbm.at[sv.at[j0 + 2]], rows_a, sem_a)
                return 0
            lax.fori_loop(0, GRP // 2, pair_body, 0)
            pltpu.make_async_copy(rows_b, acc_sh.at[dv.at[GRP - 1]],
                                  sem_wb).wait()
        plsc.subcore_barrier()
        for b in range(NODES_PER_TILE // CH):
            sl = pl.ds(base + b * CH, CH)
            pltpu.sync_copy(acc_sh.at[sl], rows_a)
            def trow(r, _):
                for v in range(D // L):
                    cs = pl.ds(v * L, L)
                    e2 = jnp.exp(rows_a[r, cs] * 2.0)
                    rows_a[r, cs] = 1.0 - 2.0 / (e2 + 1.0)
                return 0
            lax.fori_loop(0, CH, trow, 0)
            pltpu.sync_copy(rows_a, h_hbm.at[sl])
            def zrow(r, _):
                for v in range(D // L):
                    rows_a[r, pl.ds(v * L, L)] = zeros
                return 0
            lax.fori_loop(0, CH, zrow, 0)
            pltpu.sync_copy(rows_a, acc_sh.at[sl])
        plsc.subcore_barrier()
        return 0
    lax.fori_loop(0, ITERS, iter_body, 0)


def _tail_body(h_ref, abs_ref, nf_ref, ns_ref, masks_ref):
    h = h_ref[...]
    ab = abs_ref[...]
    s2 = jnp.sum(h * h, axis=1)
    iota = jax.lax.broadcasted_iota(jnp.int32, (N,), 0)
    seeds = []
    for _ in range(K):
        i = jnp.argmax(s2)
        seeds.append(h_ref[pl.ds(i, 1), :])
        s2 = jnp.where(iota == i, -jnp.inf, s2)
    seeds = jnp.concatenate(seeds, axis=0)
    logits = jax.lax.dot_general(seeds, h, (((1,), (1,)), ((), ())),
                                 preferred_element_type=jnp.float32)
    logits = logits * (1.0 / jnp.sqrt(jnp.float32(D)))
    m = jnp.max(logits, axis=1, keepdims=True)
    e = jnp.exp(logits - m)
    s = jnp.sum(e, axis=1, keepdims=True)
    masks = e / s
    masks_ref[...] = masks
    nf_ref[...] = jax.lax.dot_general(masks, ab, (((1,), (0,)), ((), ())),
                                      preferred_element_type=jnp.float32)
    ns_ref[...] = jnp.max(masks, axis=1)


@jax.jit
def _tail(h, abstract):
    return pl.pallas_call(
        _tail_body,
        out_shape=(
            jax.ShapeDtypeStruct((K, D), jnp.float32),
            jax.ShapeDtypeStruct((K,), jnp.float32),
            jax.ShapeDtypeStruct((K, N), jnp.float32),
        ),
    )(h, abstract)


def kernel(x, edge_index, init_state, Wg, bg, Wk, bk, Wq, bq):
    src = edge_index[0]
    dst = edge_index[1]
    pad = EP - E
    src_p = jnp.concatenate([src, jnp.zeros((pad,), jnp.int32)])
    dst_p = jnp.concatenate([dst, jnp.full((pad,), DUMP, jnp.int32)])
    src3 = src_p.reshape(NW, NCHUNK, CH)
    dst3 = dst_p.reshape(NW, NCHUNK, CH)
    x_pad = jnp.concatenate([x, jnp.zeros((NPAD - N, D), jnp.float32)])
    h0_pad = jnp.concatenate([init_state,
                              jnp.zeros((NPAD - N, D), jnp.float32)])
    wcat = jnp.concatenate([Wg, Wk, Wq], axis=1)
    bcat = jnp.concatenate([bg, bk, bq])

    norm_full, inv_full, normx = _deg_stage(dst_p, x_pad)
    partials = _agg_sc(src3, dst3, normx)
    abstract, kn, qn = _encode(partials, x_pad, norm_full, inv_full, wcat,
                               bcat)
    cos3, m_part = _cos_sc(src3, dst3, kn, qn)
    w3 = _w_sc(dst3, cos3, m_part)

    pad2 = EPP - EP
    src_pp = jnp.concatenate([src_p, jnp.zeros((pad2,), jnp.int32)])
    dst_pp = jnp.concatenate([dst_p, jnp.full((pad2,), DUMP, jnp.int32)])
    w_pp = jnp.concatenate([w3.reshape(EP), jnp.zeros((pad2,), jnp.float32)])
    h = _prop_sc(src_pp.reshape(NS, NCH4, CH),
                 dst_pp.reshape(NS, NCH4, CH),
                 w_pp.reshape(NS, NCH4, CH), h0_pad)
    return _tail(h[:N], abstract[:N])


# speedup vs baseline: 1.2451x; 1.2451x over previous
import functools

import jax
import jax.numpy as jnp
from jax import lax
from jax.experimental import pallas as pl
from jax.experimental.pallas import tpu as pltpu
from jax.experimental.pallas import tpu_sc as plsc

N = 10000
E = 320000
D = 128
K = 5
ITERS = 25

NPAD = 10240
DUMP = NPAD - 1
NC, NS, L = 2, 16, 16
NW = NC * NS
CH = 128
NCHUNK = 79
EPT = NCHUNK * CH
EP = EPT * NW
NODES_PER_TILE = NPAD // NS

NCHP = 80
GRP = 16
NGRP = NCHP // GRP
EPP = NW * NCHP * CH


def _sc_mesh():
    return plsc.VectorSubcoreMesh(core_axis_name="c", subcore_axis_name="s")


_SC1_MESH = plsc.VectorSubcoreMesh(core_axis_name="c", subcore_axis_name="s",
                                   num_cores=1)
_SC_PARAMS = pltpu.CompilerParams(needs_layout_passes=False)


@functools.partial(
    pl.kernel,
    out_type=jax.ShapeDtypeStruct((NC, NPAD), jnp.float32),
    mesh=_sc_mesh(),
    scratch_types=[
        pltpu.VMEM((EPT,), jnp.int32),
        pltpu.VMEM((NPAD,), jnp.float32),
        pltpu.VMEM((NODES_PER_TILE,), jnp.float32),
        pltpu.VMEM((NODES_PER_TILE,), jnp.float32),
        pltpu.VMEM_SHARED((NS, NPAD), jnp.float32),
    ],
    compiler_params=_SC_PARAMS,
)
def _deg_sc(dst_hbm, out_hbm, dst_v, hist_v, acc_v, tmp_v, stage_sh):
    cid = lax.axis_index("c")
    sid = lax.axis_index("s")
    wid = cid * NS + sid
    zeros = jnp.zeros((L,), jnp.float32)
    ones = jnp.ones((L,), jnp.float32)

    def zero_body(i, _):
        hist_v[pl.ds(i * L, L)] = zeros
        return 0
    lax.fori_loop(0, NPAD // L, zero_body, 0)

    pltpu.sync_copy(dst_hbm.at[pl.ds(wid * EPT, EPT)], dst_v)

    def scat_body(i, _):
        idx = dst_v[pl.ds(i * L, L)]
        plsc.addupdate_scatter(hist_v, [idx], ones)
        return 0
    lax.fori_loop(0, EPT // L, scat_body, 0)

    pltpu.sync_copy(hist_v, stage_sh.at[sid])
    plsc.subcore_barrier()

    base = sid * NODES_PER_TILE
    def merge_zero(i, _):
        acc_v[pl.ds(i * L, L)] = zeros
        return 0
    lax.fori_loop(0, NODES_PER_TILE // L, merge_zero, 0)
    for t in range(NS):
        pltpu.sync_copy(stage_sh.at[t, pl.ds(base, NODES_PER_TILE)], tmp_v)
        def add_body(i, _):
            acc_v[pl.ds(i * L, L)] = acc_v[pl.ds(i * L, L)] + tmp_v[pl.ds(i * L, L)]
            return 0
        lax.fori_loop(0, NODES_PER_TILE // L, add_body, 0)
    pltpu.sync_copy(acc_v, out_hbm.at[cid, pl.ds(base, NODES_PER_TILE)])


def _degfix_body(p0_ref, p1_ref, x_ref, norm_ref, inv_ref, normx_ref):
    deg = p0_ref[...] + p1_ref[...] + 1.0
    nrm = jax.lax.rsqrt(deg)
    norm_ref[...] = nrm
    inv_ref[...] = 1.0 / deg
    normx_ref[...] = x_ref[...] * nrm


@jax.jit
def _deg_stage(dst, x_pad):
    partials = _deg_sc(dst)
    norm, inv, normx = pl.pallas_call(
        _degfix_body,
        out_shape=(
            jax.ShapeDtypeStruct((NPAD, 1), jnp.float32),
            jax.ShapeDtypeStruct((NPAD, 1), jnp.float32),
            jax.ShapeDtypeStruct((NPAD, D), jnp.float32),
        ),
    )(partials[0][:, None], partials[1][:, None], x_pad)
    return norm.reshape(NPAD), inv.reshape(NPAD), normx


@functools.partial(
    pl.kernel,
    out_type=jax.ShapeDtypeStruct((NC, NPAD, D), jnp.float32),
    mesh=_sc_mesh(),
    scratch_types=[
        pltpu.VMEM((NCHUNK, CH), jnp.int32),
        pltpu.VMEM((NCHUNK, CH), jnp.int32),
        pltpu.VMEM((CH, D), jnp.float32),
        pltpu.VMEM_SHARED((NPAD, D), jnp.float32),
        pltpu.SemaphoreType.DMA,
    ],
    compiler_params=_SC_PARAMS,
)
def _agg_sc(src_hbm, dst_hbm, nx_hbm, out_hbm,
            src_v, dst_v, rows_a, acc_sh, sem_a):
    cid = lax.axis_index("c")
    sid = lax.axis_index("s")
    wid = cid * NS + sid
    zeros = jnp.zeros((L,), jnp.float32)

    pltpu.sync_copy(src_hbm.at[wid], src_v)
    pltpu.sync_copy(dst_hbm.at[wid], dst_v)

    def z_rows(r, _):
        for v in range(D // L):
            rows_a[r, pl.ds(v * L, L)] = zeros
        return 0
    lax.fori_loop(0, CH, z_rows, 0)
    base = sid * NODES_PER_TILE
    for b in range(NODES_PER_TILE // CH):
        pltpu.sync_copy(rows_a, acc_sh.at[pl.ds(base + b * CH, CH)])
    plsc.subcore_barrier()

    def chunk_body(j, _):
        pltpu.async_copy(nx_hbm.at[src_v.at[j]], rows_a, sem_a).wait()
        pltpu.sync_copy(rows_a, acc_sh.at[dst_v.at[j]], add=True)
        return 0
    lax.fori_loop(0, NCHUNK, chunk_body, 0)

    plsc.subcore_barrier()
    pltpu.sync_copy(acc_sh.at[pl.ds(base, NODES_PER_TILE)],
                    out_hbm.at[cid, pl.ds(base, NODES_PER_TILE)])


ROWS_BLK = 1280


def _enc_body(p0_ref, p1_ref, x_ref, norm_ref, inv_ref, w_ref, b_ref,
              abs_ref, kn_ref, qn_ref):
    i = pl.program_id(0)
    xagg = (norm_ref[...] * (p0_ref[...] + p1_ref[...])
            + x_ref[...] * inv_ref[...])
    y = jax.lax.dot_general(xagg, w_ref[...], (((1,), (0,)), ((), ())),
                            preferred_element_type=jnp.float32) + b_ref[...]
    ab, kk, qq = y[:, :D], y[:, D:2 * D], y[:, 2 * D:]
    row = jax.lax.broadcasted_iota(jnp.int32, (ROWS_BLK, 1), 0) + i * ROWS_BLK
    valid = row < N
    nk = jnp.sqrt(jnp.sum(kk * kk, axis=1, keepdims=True))
    nq = jnp.sqrt(jnp.sum(qq * qq, axis=1, keepdims=True))
    kn = jnp.where(valid, kk / jnp.maximum(nk, 1e-8), 0.0)
    qn = jnp.where(valid, qq / jnp.maximum(nq, 1e-8), 0.0)
    abs_ref[...] = ab
    kn_ref[...] = kn
    qn_ref[...] = qn


@jax.jit
def _encode(p, x_pad, norm_full, inv_full, wcat, bcat):
    grid = NPAD // ROWS_BLK
    blk = lambda i: (i, 0)
    return pl.pallas_call(
        _enc_body,
        grid=(grid,),
        in_specs=[
            pl.BlockSpec((ROWS_BLK, D), blk),
            pl.BlockSpec((ROWS_BLK, D), blk),
            pl.BlockSpec((ROWS_BLK, D), blk),
            pl.BlockSpec((ROWS_BLK, 1), blk),
            pl.BlockSpec((ROWS_BLK, 1), blk),
            pl.BlockSpec((D, 3 * D), lambda i: (0, 0)),
            pl.BlockSpec((1, 3 * D), lambda i: (0, 0)),
        ],
        out_specs=[
            pl.BlockSpec((ROWS_BLK, D), blk),
            pl.BlockSpec((ROWS_BLK, D), blk),
            pl.BlockSpec((ROWS_BLK, D), blk),
        ],
        out_shape=[
            jax.ShapeDtypeStruct((NPAD, D), jnp.float32),
            jax.ShapeDtypeStruct((NPAD, D), jnp.float32),
            jax.ShapeDtypeStruct((NPAD, D), jnp.float32),
        ],
    )(p[0], p[1], x_pad, norm_full[:, None], inv_full[:, None], wcat,
      bcat[None, :])


@functools.partial(
    pl.kernel,
    out_type=(
        jax.ShapeDtypeStruct((NW, NCHUNK, CH), jnp.float32),
        jax.ShapeDtypeStruct((NC, NPAD), jnp.float32),
    ),
    mesh=_sc_mesh(),
    scratch_types=[
        pltpu.VMEM((NCHUNK, CH), jnp.int32),
        pltpu.VMEM((NCHUNK, CH), jnp.int32),
        pltpu.VMEM((NCHUNK, CH), jnp.float32),
        pltpu.VMEM((NPAD,), jnp.float32),
        pltpu.VMEM((NODES_PER_TILE,), jnp.float32),
        pltpu.VMEM((CH, D), jnp.float32),
        pltpu.VMEM((CH, D), jnp.float32),
        pltpu.VMEM_SHARED((NS, NPAD), jnp.float32),
        pltpu.SemaphoreType.DMA,
        pltpu.SemaphoreType.DMA,
    ],
    compiler_params=_SC_PARAMS,
)
def _cos_sc(src_hbm, dst_hbm, kn_hbm, qn_hbm, cos_hbm, m_hbm,
            src_v, dst_v, cos_v, m_v, tmp_v, rows_k, rows_q, stage_sh,
            sem_k, sem_q):
    cid = lax.axis_index("c")
    sid = lax.axis_index("s")
    wid = cid * NS + sid
    neg_inf = jnp.full((L,), -jnp.inf, jnp.float32)

    pltpu.sync_copy(src_hbm.at[wid], src_v)
    pltpu.sync_copy(dst_hbm.at[wid], dst_v)

    def minit(i, _):
        m_v[pl.ds(i * L, L)] = neg_inf
        return 0
    lax.fori_loop(0, NPAD // L, minit, 0)

    lane = lax.iota(jnp.int32, L)

    def chunk_body(j, _):
        ck = pltpu.async_copy(kn_hbm.at[src_v.at[j]], rows_k, sem_k)
        cq = pltpu.async_copy(qn_hbm.at[dst_v.at[j]], rows_q, sem_q)
        ck.wait()
        cq.wait()
        def group_body(t, _):
            ri = t * L + lane
            def col_body(v4, acc):
                for u in range(4):
                    vb = jnp.full((L,), v4 * 4 + u, jnp.int32)
                    ka = plsc.load_gather(rows_k, [ri, vb])
                    qa = plsc.load_gather(rows_q, [ri, vb])
                    acc = acc + ka * qa
                return acc
            dots = lax.fori_loop(0, D // 4, col_body,
                                 jnp.zeros((L,), jnp.float32))
            cos_v[j, pl.ds(t * L, L)] = dots
            d16 = dst_v[j, pl.ds(t * L, L)]
            rank, _last = plsc.scan_count(d16)
            def round_body(k, _):
                mg = plsc.load_gather(m_v, [d16])
                mx = jnp.maximum(mg, dots)
                plsc.store_scatter(m_v, [d16], mx, mask=rank == k)
                return 0
            lax.fori_loop(0, L + 1, round_body, 0)
            return 0
        lax.fori_loop(0, CH // L, group_body, 0)
        return 0
    lax.fori_loop(0, NCHUNK, chunk_body, 0)

    pltpu.sync_copy(cos_v, cos_hbm.at[wid])
    pltpu.sync_copy(m_v, stage_sh.at[sid])
    plsc.subcore_barrier()

    base = sid * NODES_PER_TILE
    def mz(i, _):
        tmp_v[pl.ds(i * L, L)] = neg_inf
        return 0
    lax.fori_loop(0, NODES_PER_TILE // L, mz, 0)
    for t in range(NS):
        pltpu.sync_copy(stage_sh.at[t, pl.ds(base, NODES_PER_TILE)],
                        m_v.at[pl.ds(0, NODES_PER_TILE)])
        def mmax(i, _):
            sl = pl.ds(i * L, L)
            tmp_v[sl] = jnp.maximum(tmp_v[sl], m_v[sl])
            return 0
        lax.fori_loop(0, NODES_PER_TILE // L, mmax, 0)
    pltpu.sync_copy(tmp_v, m_hbm.at[cid, pl.ds(base, NODES_PER_TILE)])


@functools.partial(
    pl.kernel,
    out_type=jax.ShapeDtypeStruct((NW, NCHUNK, CH), jnp.float32),
    mesh=_sc_mesh(),
    scratch_types=[
        pltpu.VMEM((NCHUNK, CH), jnp.int32),
        pltpu.VMEM((NCHUNK, CH), jnp.float32),
        pltpu.VMEM((NPAD,), jnp.float32),
        pltpu.VMEM((NPAD,), jnp.float32),
    ],
    compiler_params=_SC_PARAMS,
)
def _w_sc(dst_hbm, cos_hbm, m_hbm, w_hbm, dst_v, w_v, m_v, tmp_v):
    cid = lax.axis_index("c")
    sid = lax.axis_index("s")
    wid = cid * NS + sid

    pltpu.sync_copy(dst_hbm.at[wid], dst_v)
    pltpu.sync_copy(cos_hbm.at[wid], w_v)
    pltpu.sync_copy(m_hbm.at[0], m_v)
    pltpu.sync_copy(m_hbm.at[1], tmp_v)

    def mmerge(i, _):
        sl = pl.ds(i * L, L)
        m_v[sl] = jnp.maximum(m_v[sl], tmp_v[sl])
        return 0
    lax.fori_loop(0, NPAD // L, mmerge, 0)

    def wchunk(j, _):
        def winner(t, _):
            sl = pl.ds(t * L, L)
            mg = plsc.load_gather(m_v, [dst_v[j, sl]])
            w_v[j, sl] = jnp.exp(w_v[j, sl] - mg)
            return 0
        return lax.fori_loop(0, CH // L, winner, 0)
    lax.fori_loop(0, NCHUNK, wchunk, 0)
    pltpu.sync_copy(w_v, w_hbm.at[wid])


def _make_prop_step(first):
    @functools.partial(
        pl.kernel,
        out_type=(
            jax.ShapeDtypeStruct((NC, NPAD, D), jnp.float32),
            jax.ShapeDtypeStruct((NC, NPAD, D), jnp.float32),
        ),
        mesh=_sc_mesh(),
        scratch_types=[
            pltpu.VMEM((GRP, CH), jnp.int32),
            pltpu.VMEM((GRP, CH), jnp.int32),
            pltpu.VMEM((GRP, CH), jnp.int32),
            pltpu.VMEM((GRP, CH), jnp.int32),
            pltpu.VMEM((GRP, CH), jnp.float32),
            pltpu.VMEM((GRP, CH), jnp.float32),
            pltpu.VMEM((CH, D), jnp.float32),
            pltpu.VMEM((CH, D), jnp.float32),
            pltpu.VMEM_SHARED((NPAD, D), jnp.float32),
            pltpu.SemaphoreType.DMA,
            pltpu.SemaphoreType.DMA,
            pltpu.SemaphoreType.DMA,
            pltpu.SemaphoreType.DMA,
            pltpu.SemaphoreType.DMA,
            pltpu.SemaphoreType.DMA,
        ],
        compiler_params=_SC_PARAMS,
    )
    def step(src_hbm, dst_hbm, w_hbm, hin_hbm, pout_hbm, htab_hbm,
             src0_v, src1_v, dst0_v, dst1_v, w0_v, w1_v, rows_a, rows_b,
             acc_sh, sem_a, sem_b, sem_i0, sem_i1, sem_wa, sem_wb):
        cid = lax.axis_index("c")
        sid = lax.axis_index("s")
        wid = cid * NS + sid
        base = sid * NODES_PER_TILE
        zeros = jnp.zeros((L,), jnp.float32)
        sbuf = (src0_v, src1_v)
        dbuf = (dst0_v, dst1_v)
        wbuf = (w0_v, w1_v)
        ibuf_sem = (sem_i0, sem_i1)
        htab = htab_hbm.at[cid]

        for b in range(NODES_PER_TILE // CH):
            sl = pl.ds(base + b * CH, CH)
            if first:
                pltpu.sync_copy(hin_hbm.at[sl], rows_a)
            else:
                pltpu.sync_copy(hin_hbm.at[0, sl], rows_a)
                pltpu.sync_copy(hin_hbm.at[1, sl], rows_b)
                def tanh_row(r, _):
                    for v in range(D // L):
                        cs = pl.ds(v * L, L)
                        s = rows_a[r, cs] + rows_b[r, cs]
                        e2 = jnp.exp(s * 2.0)
                        rows_a[r, cs] = 1.0 - 2.0 / (e2 + 1.0)
                    return 0
                lax.fori_loop(0, CH, tanh_row, 0)
            pltpu.sync_copy(rows_a, htab.at[sl])
        def z_rows(r, _):
            for v in range(D // L):
                rows_a[r, pl.ds(v * L, L)] = zeros
            return 0
        lax.fori_loop(0, CH, z_rows, 0)
        for b in range(NODES_PER_TILE // CH):
            pltpu.sync_copy(rows_a, acc_sh.at[pl.ds(base + b * CH, CH)])
        plsc.subcore_barrier()

        def issue_grp(g, b):
            gs = pl.ds(g * GRP, GRP)
            pltpu.async_copy(src_hbm.at[wid, gs], sbuf[b], ibuf_sem[b])
            pltpu.async_copy(dst_hbm.at[wid, gs], dbuf[b], ibuf_sem[b])
            pltpu.async_copy(w_hbm.at[wid, gs], wbuf[b], ibuf_sem[b])

        def wait_grp(g, b):
            gs = pl.ds(g * GRP, GRP)
            pltpu.make_async_copy(src_hbm.at[wid, gs], sbuf[b],
                                  ibuf_sem[b]).wait()
            pltpu.make_async_copy(dst_hbm.at[wid, gs], dbuf[b],
                                  ibuf_sem[b]).wait()
            pltpu.make_async_copy(w_hbm.at[wid, gs], wbuf[b],
                                  ibuf_sem[b]).wait()

        def scale(wv, rows, jj):
            jb = jnp.full((L,), jj, jnp.int32)
            def scale4(ei, _):
                for u in range(4):
                    e = ei * 4 + u
                    wb = plsc.load_gather(
                        wv, [jb, jnp.full((L,), e, jnp.int32)])
                    for v in range(D // L):
                        sl = pl.ds(v * L, L)
                        rows[e, sl] = rows[e, sl] * wb
                return 0
            lax.fori_loop(0, CH // 4, scale4, 0)

        issue_grp(0, 0)
        for g in range(NGRP):
            b = g % 2
            sv, dv, wv = sbuf[b], dbuf[b], wbuf[b]
            wait_grp(g, b)
            if g + 1 < NGRP:
                issue_grp(g + 1, 1 - b)
            pltpu.async_copy(htab.at[sv.at[0]], rows_a, sem_a)
            def pair_body(p, _):
                j0 = 2 * p
                @pl.when(j0 >= 1)
                def _():
                    pltpu.make_async_copy(rows_b, acc_sh.at[dv.at[j0 - 1]],
                                          sem_wb).wait()
                pltpu.async_copy(htab.at[sv.at[j0 + 1]], rows_b, sem_b)
                pltpu.make_async_copy(htab.at[sv.at[j0]], rows_a,
                                      sem_a).wait()
                scale(wv, rows_a, j0)
                pltpu.async_copy(rows_a, acc_sh.at[dv.at[j0]], sem_wa,
                                 add=True)
                pltpu.make_async_copy(htab.at[sv.at[j0 + 1]], rows_b,
                                      sem_b).wait()
                scale(wv, rows_b, j0 + 1)
                pltpu.async_copy(rows_b, acc_sh.at[dv.at[j0 + 1]], sem_wb,
                                 add=True)
                pltpu.make_async_copy(rows_a, acc_sh.at[dv.at[j0]],
                                      sem_wa).wait()
                @pl.when(p < GRP // 2 - 1)
                def _():
                    pltpu.async_copy(htab.at[sv.at[j0 + 2]], rows_a, sem_a)
                return 0
            lax.fori_loop(0, GRP // 2, pair_body, 0)
            pltpu.make_async_copy(rows_b, acc_sh.at[dv.at[GRP - 1]],
                                  sem_wb).wait()
        plsc.subcore_barrier()
        pltpu.sync_copy(acc_sh.at[pl.ds(base, NODES_PER_TILE)],
                        pout_hbm.at[cid, pl.ds(base, NODES_PER_TILE)])
    return step


_prop_first = _make_prop_step(True)
_prop_next = _make_prop_step(False)


def _tail_body(p0_ref, p1_ref, abs_ref, nf_ref, ns_ref, masks_ref, h_ref):
    h_ref[...] = jnp.tanh(p0_ref[...] + p1_ref[...])
    h = h_ref[...]
    ab = abs_ref[...]
    s2 = jnp.sum(h * h, axis=1)
    iota = jax.lax.broadcasted_iota(jnp.int32, (N,), 0)
    seeds = []
    for _ in range(K):
        i = jnp.argmax(s2)
        seeds.append(h_ref[pl.ds(i, 1), :])
        s2 = jnp.where(iota == i, -jnp.inf, s2)
    seeds = jnp.concatenate(seeds, axis=0)
    logits = jax.lax.dot_general(seeds, h, (((1,), (1,)), ((), ())),
                                 preferred_element_type=jnp.float32)
    logits = logits * (1.0 / jnp.sqrt(jnp.float32(D)))
    m = jnp.max(logits, axis=1, keepdims=True)
    e = jnp.exp(logits - m)
    s = jnp.sum(e, axis=1, keepdims=True)
    masks = e / s
    masks_ref[...] = masks
    nf_ref[...] = jax.lax.dot_general(masks, ab, (((1,), (0,)), ((), ())),
                                      preferred_element_type=jnp.float32)
    ns_ref[...] = jnp.max(masks, axis=1)


@jax.jit
def _tail(p0, p1, abstract):
    nf, ns, masks, _h = pl.pallas_call(
        _tail_body,
        out_shape=(
            jax.ShapeDtypeStruct((K, D), jnp.float32),
            jax.ShapeDtypeStruct((K,), jnp.float32),
            jax.ShapeDtypeStruct((K, N), jnp.float32),
            jax.ShapeDtypeStruct((N, D), jnp.float32),
        ),
    )(p0, p1, abstract)
    return nf, ns, masks


def kernel(x, edge_index, init_state, Wg, bg, Wk, bk, Wq, bq):
    src = edge_index[0]
    dst = edge_index[1]
    pad = EP - E
    src_p = jnp.concatenate([src, jnp.zeros((pad,), jnp.int32)])
    dst_p = jnp.concatenate([dst, jnp.full((pad,), DUMP, jnp.int32)])
    src3 = src_p.reshape(NW, NCHUNK, CH)
    dst3 = dst_p.reshape(NW, NCHUNK, CH)
    x_pad = jnp.concatenate([x, jnp.zeros((NPAD - N, D), jnp.float32)])
    h0_pad = jnp.concatenate([init_state,
                              jnp.zeros((NPAD - N, D), jnp.float32)])
    wcat = jnp.concatenate([Wg, Wk, Wq], axis=1)
    bcat = jnp.concatenate([bg, bk, bq])

    norm_full, inv_full, normx = _deg_stage(dst_p, x_pad)
    partials = _agg_sc(src3, dst3, normx)
    abstract, kn, qn = _encode(partials, x_pad, norm_full, inv_full, wcat,
                               bcat)
    cos3, m_part = _cos_sc(src3, dst3, kn, qn)
    w3 = _w_sc(dst3, cos3, m_part)

    pad2 = EPP - EP
    src_pp = jnp.concatenate([src_p, jnp.zeros((pad2,), jnp.int32)])
    dst_pp = jnp.concatenate([dst_p, jnp.full((pad2,), DUMP, jnp.int32)])
    w_pp = jnp.concatenate([w3.reshape(EP), jnp.zeros((pad2,), jnp.float32)])
    src4 = src_pp.reshape(NW, NCHP, CH)
    dst4 = dst_pp.reshape(NW, NCHP, CH)
    w4 = w_pp.reshape(NW, NCHP, CH)
    p, _ht = _prop_first(src4, dst4, w4, h0_pad)
    for _ in range(ITERS - 1):
        p, _ht = _prop_next(src4, dst4, w4, p)
    return _tail(p[0][:N], p[1][:N], abstract[:N])


# speedup vs baseline: 1.2454x; 1.0002x over previous
import functools

import jax
import jax.numpy as jnp
from jax import lax
from jax.experimental import pallas as pl
from jax.experimental.pallas import tpu as pltpu
from jax.experimental.pallas import tpu_sc as plsc

N = 10000
E = 320000
D = 128
K = 5
ITERS = 25

NPAD = 10240
DUMP = NPAD - 1
NC, NS, L = 2, 16, 16
NW = NC * NS
CH = 128
NCHUNK = 79
EPT = NCHUNK * CH
EP = EPT * NW
NODES_PER_TILE = NPAD // NS

NCHP = 80
GRP = 16
NGRP = NCHP // GRP
EPP = NW * NCHP * CH


def _sc_mesh():
    return plsc.VectorSubcoreMesh(core_axis_name="c", subcore_axis_name="s")


_SC1_MESH = plsc.VectorSubcoreMesh(core_axis_name="c", subcore_axis_name="s",
                                   num_cores=1)
_SC_PARAMS = pltpu.CompilerParams(needs_layout_passes=False)


@functools.partial(
    pl.kernel,
    out_type=jax.ShapeDtypeStruct((NC, NPAD), jnp.float32),
    mesh=_sc_mesh(),
    scratch_types=[
        pltpu.VMEM((EPT,), jnp.int32),
        pltpu.VMEM((NPAD,), jnp.float32),
        pltpu.VMEM((NODES_PER_TILE,), jnp.float32),
        pltpu.VMEM((NODES_PER_TILE,), jnp.float32),
        pltpu.VMEM_SHARED((NS, NPAD), jnp.float32),
    ],
    compiler_params=_SC_PARAMS,
)
def _deg_sc(dst_hbm, out_hbm, dst_v, hist_v, acc_v, tmp_v, stage_sh):
    cid = lax.axis_index("c")
    sid = lax.axis_index("s")
    wid = cid * NS + sid
    zeros = jnp.zeros((L,), jnp.float32)
    ones = jnp.ones((L,), jnp.float32)

    def zero_body(i, _):
        hist_v[pl.ds(i * L, L)] = zeros
        return 0
    lax.fori_loop(0, NPAD // L, zero_body, 0)

    pltpu.sync_copy(dst_hbm.at[pl.ds(wid * EPT, EPT)], dst_v)

    def scat_body(i, _):
        idx = dst_v[pl.ds(i * L, L)]
        plsc.addupdate_scatter(hist_v, [idx], ones)
        return 0
    lax.fori_loop(0, EPT // L, scat_body, 0)

    pltpu.sync_copy(hist_v, stage_sh.at[sid])
    plsc.subcore_barrier()

    base = sid * NODES_PER_TILE
    def merge_zero(i, _):
        acc_v[pl.ds(i * L, L)] = zeros
        return 0
    lax.fori_loop(0, NODES_PER_TILE // L, merge_zero, 0)
    for t in range(NS):
        pltpu.sync_copy(stage_sh.at[t, pl.ds(base, NODES_PER_TILE)], tmp_v)
        def add_body(i, _):
            acc_v[pl.ds(i * L, L)] = acc_v[pl.ds(i * L, L)] + tmp_v[pl.ds(i * L, L)]
            return 0
        lax.fori_loop(0, NODES_PER_TILE // L, add_body, 0)
    pltpu.sync_copy(acc_v, out_hbm.at[cid, pl.ds(base, NODES_PER_TILE)])


def _degfix_body(p0_ref, p1_ref, x_ref, norm_ref, inv_ref, normx_ref):
    deg = p0_ref[...] + p1_ref[...] + 1.0
    nrm = jax.lax.rsqrt(deg)
    norm_ref[...] = nrm
    inv_ref[...] = 1.0 / deg
    normx_ref[...] = x_ref[...] * nrm


@jax.jit
def _deg_stage(dst, x_pad):
    partials = _deg_sc(dst)
    norm, inv, normx = pl.pallas_call(
        _degfix_body,
        out_shape=(
            jax.ShapeDtypeStruct((NPAD, 1), jnp.float32),
            jax.ShapeDtypeStruct((NPAD, 1), jnp.float32),
            jax.ShapeDtypeStruct((NPAD, D), jnp.float32),
        ),
    )(partials[0][:, None], partials[1][:, None], x_pad)
    return norm.reshape(NPAD), inv.reshape(NPAD), normx


@functools.partial(
    pl.kernel,
    out_type=jax.ShapeDtypeStruct((NC, NPAD, D), jnp.float32),
    mesh=_sc_mesh(),
    scratch_types=[
        pltpu.VMEM((NCHUNK, CH), jnp.int32),
        pltpu.VMEM((NCHUNK, CH), jnp.int32),
        pltpu.VMEM((CH, D), jnp.float32),
        pltpu.VMEM_SHARED((NPAD, D), jnp.float32),
        pltpu.SemaphoreType.DMA,
    ],
    compiler_params=_SC_PARAMS,
)
def _agg_sc(src_hbm, dst_hbm, nx_hbm, out_hbm,
            src_v, dst_v, rows_a, acc_sh, sem_a):
    cid = lax.axis_index("c")
    sid = lax.axis_index("s")
    wid = cid * NS + sid
    zeros = jnp.zeros((L,), jnp.float32)

    pltpu.sync_copy(src_hbm.at[wid], src_v)
    pltpu.sync_copy(dst_hbm.at[wid], dst_v)

    def z_rows(r, _):
        for v in range(D // L):
            rows_a[r, pl.ds(v * L, L)] = zeros
        return 0
    lax.fori_loop(0, CH, z_rows, 0)
    base = sid * NODES_PER_TILE
    for b in range(NODES_PER_TILE // CH):
        pltpu.sync_copy(rows_a, acc_sh.at[pl.ds(base + b * CH, CH)])
    plsc.subcore_barrier()

    def chunk_body(j, _):
        pltpu.async_copy(nx_hbm.at[src_v.at[j]], rows_a, sem_a).wait()
        pltpu.sync_copy(rows_a, acc_sh.at[dst_v.at[j]], add=True)
        return 0
    lax.fori_loop(0, NCHUNK, chunk_body, 0)

    plsc.subcore_barrier()
    pltpu.sync_copy(acc_sh.at[pl.ds(base, NODES_PER_TILE)],
                    out_hbm.at[cid, pl.ds(base, NODES_PER_TILE)])


ROWS_BLK = 1280


def _enc_body(p0_ref, p1_ref, x_ref, norm_ref, inv_ref, w_ref, b_ref,
              abs_ref, kn_ref, qn_ref):
    i = pl.program_id(0)
    xagg = (norm_ref[...] * (p0_ref[...] + p1_ref[...])
            + x_ref[...] * inv_ref[...])
    y = jax.lax.dot_general(xagg, w_ref[...], (((1,), (0,)), ((), ())),
                            preferred_element_type=jnp.float32) + b_ref[...]
    ab, kk, qq = y[:, :D], y[:, D:2 * D], y[:, 2 * D:]
    row = jax.lax.broadcasted_iota(jnp.int32, (ROWS_BLK, 1), 0) + i * ROWS_BLK
    valid = row < N
    nk = jnp.sqrt(jnp.sum(kk * kk, axis=1, keepdims=True))
    nq = jnp.sqrt(jnp.sum(qq * qq, axis=1, keepdims=True))
    kn = jnp.where(valid, kk / jnp.maximum(nk, 1e-8), 0.0)
    qn = jnp.where(valid, qq / jnp.maximum(nq, 1e-8), 0.0)
    abs_ref[...] = ab
    kn_ref[...] = kn
    qn_ref[...] = qn


@jax.jit
def _encode(p, x_pad, norm_full, inv_full, wcat, bcat):
    grid = NPAD // ROWS_BLK
    blk = lambda i: (i, 0)
    return pl.pallas_call(
        _enc_body,
        grid=(grid,),
        in_specs=[
            pl.BlockSpec((ROWS_BLK, D), blk),
            pl.BlockSpec((ROWS_BLK, D), blk),
            pl.BlockSpec((ROWS_BLK, D), blk),
            pl.BlockSpec((ROWS_BLK, 1), blk),
            pl.BlockSpec((ROWS_BLK, 1), blk),
            pl.BlockSpec((D, 3 * D), lambda i: (0, 0)),
            pl.BlockSpec((1, 3 * D), lambda i: (0, 0)),
        ],
        out_specs=[
            pl.BlockSpec((ROWS_BLK, D), blk),
            pl.BlockSpec((ROWS_BLK, D), blk),
            pl.BlockSpec((ROWS_BLK, D), blk),
        ],
        out_shape=[
            jax.ShapeDtypeStruct((NPAD, D), jnp.float32),
            jax.ShapeDtypeStruct((NPAD, D), jnp.float32),
            jax.ShapeDtypeStruct((NPAD, D), jnp.float32),
        ],
    )(p[0], p[1], x_pad, norm_full[:, None], inv_full[:, None], wcat,
      bcat[None, :])


@functools.partial(
    pl.kernel,
    out_type=(
        jax.ShapeDtypeStruct((NW, NCHUNK, CH), jnp.float32),
        jax.ShapeDtypeStruct((NC, NPAD), jnp.float32),
    ),
    mesh=_sc_mesh(),
    scratch_types=[
        pltpu.VMEM((NCHUNK, CH), jnp.int32),
        pltpu.VMEM((NCHUNK, CH), jnp.int32),
        pltpu.VMEM((NCHUNK, CH), jnp.float32),
        pltpu.VMEM((NPAD,), jnp.float32),
        pltpu.VMEM((NODES_PER_TILE,), jnp.float32),
        pltpu.VMEM((CH, D), jnp.float32),
        pltpu.VMEM((CH, D), jnp.float32),
        pltpu.VMEM_SHARED((NS, NPAD), jnp.float32),
        pltpu.SemaphoreType.DMA,
        pltpu.SemaphoreType.DMA,
    ],
    compiler_params=_SC_PARAMS,
)
def _cos_sc(src_hbm, dst_hbm, kn_hbm, qn_hbm, cos_hbm, m_hbm,
            src_v, dst_v, cos_v, m_v, tmp_v, rows_k, rows_q, stage_sh,
            sem_k, sem_q):
    cid = lax.axis_index("c")
    sid = lax.axis_index("s")
    wid = cid * NS + sid
    neg_inf = jnp.full((L,), -jnp.inf, jnp.float32)

    pltpu.sync_copy(src_hbm.at[wid], src_v)
    pltpu.sync_copy(dst_hbm.at[wid], dst_v)

    def minit(i, _):
        m_v[pl.ds(i * L, L)] = neg_inf
        return 0
    lax.fori_loop(0, NPAD // L, minit, 0)

    lane = lax.iota(jnp.int32, L)

    def chunk_body(j, _):
        ck = pltpu.async_copy(kn_hbm.at[src_v.at[j]], rows_k, sem_k)
        cq = pltpu.async_copy(qn_hbm.at[dst_v.at[j]], rows_q, sem_q)
        ck.wait()
        cq.wait()
        def group_body(t, _):
            ri = t * L + lane
            def col_body(v4, accs):
                out = []
                for u in range(4):
                    vb = jnp.full((L,), v4 * 4 + u, jnp.int32)
                    ka = plsc.load_gather(rows_k, [ri, vb])
                    qa = plsc.load_gather(rows_q, [ri, vb])
                    out.append(accs[u] + ka * qa)
                return tuple(out)
            z16 = jnp.zeros((L,), jnp.float32)
            a0, a1, a2, a3 = lax.fori_loop(0, D // 4, col_body,
                                           (z16, z16, z16, z16))
            dots = (a0 + a1) + (a2 + a3)
            cos_v[j, pl.ds(t * L, L)] = dots
            d16 = dst_v[j, pl.ds(t * L, L)]
            rank, _last = plsc.scan_count(d16)
            def round_body(k, _):
                mg = plsc.load_gather(m_v, [d16])
                mx = jnp.maximum(mg, dots)
                plsc.store_scatter(m_v, [d16], mx, mask=rank == k)
                return 0
            lax.fori_loop(0, L + 1, round_body, 0)
            return 0
        lax.fori_loop(0, CH // L, group_body, 0)
        return 0
    lax.fori_loop(0, NCHUNK, chunk_body, 0)

    pltpu.sync_copy(cos_v, cos_hbm.at[wid])
    pltpu.sync_copy(m_v, stage_sh.at[sid])
    plsc.subcore_barrier()

    base = sid * NODES_PER_TILE
    def mz(i, _):
        tmp_v[pl.ds(i * L, L)] = neg_inf
        return 0
    lax.fori_loop(0, NODES_PER_TILE // L, mz, 0)
    for t in range(NS):
        pltpu.sync_copy(stage_sh.at[t, pl.ds(base, NODES_PER_TILE)],
                        m_v.at[pl.ds(0, NODES_PER_TILE)])
        def mmax(i, _):
            sl = pl.ds(i * L, L)
            tmp_v[sl] = jnp.maximum(tmp_v[sl], m_v[sl])
            return 0
        lax.fori_loop(0, NODES_PER_TILE // L, mmax, 0)
    pltpu.sync_copy(tmp_v, m_hbm.at[cid, pl.ds(base, NODES_PER_TILE)])


@functools.partial(
    pl.kernel,
    out_type=jax.ShapeDtypeStruct((NW, NCHUNK, CH), jnp.float32),
    mesh=_sc_mesh(),
    scratch_types=[
        pltpu.VMEM((NCHUNK, CH), jnp.int32),
        pltpu.VMEM((NCHUNK, CH), jnp.float32),
        pltpu.VMEM((NPAD,), jnp.float32),
        pltpu.VMEM((NPAD,), jnp.float32),
    ],
    compiler_params=_SC_PARAMS,
)
def _w_sc(dst_hbm, cos_hbm, m_hbm, w_hbm, dst_v, w_v, m_v, tmp_v):
    cid = lax.axis_index("c")
    sid = lax.axis_index("s")
    wid = cid * NS + sid

    pltpu.sync_copy(dst_hbm.at[wid], dst_v)
    pltpu.sync_copy(cos_hbm.at[wid], w_v)
    pltpu.sync_copy(m_hbm.at[0], m_v)
    pltpu.sync_copy(m_hbm.at[1], tmp_v)

    def mmerge(i, _):
        sl = pl.ds(i * L, L)
        m_v[sl] = jnp.maximum(m_v[sl], tmp_v[sl])
        return 0
    lax.fori_loop(0, NPAD // L, mmerge, 0)

    def wchunk(j, _):
        def winner(t, _):
            sl = pl.ds(t * L, L)
            mg = plsc.load_gather(m_v, [dst_v[j, sl]])
            w_v[j, sl] = jnp.exp(w_v[j, sl] - mg)
            return 0
        return lax.fori_loop(0, CH // L, winner, 0)
    lax.fori_loop(0, NCHUNK, wchunk, 0)
    pltpu.sync_copy(w_v, w_hbm.at[wid])


def _make_prop_step(first):
    @functools.partial(
        pl.kernel,
        out_type=(
            jax.ShapeDtypeStruct((NC, NPAD, D), jnp.float32),
            jax.ShapeDtypeStruct((NC, NPAD, D), jnp.float32),
        ),
        mesh=_sc_mesh(),
        scratch_types=[
            pltpu.VMEM((GRP, CH), jnp.int32),
            pltpu.VMEM((GRP, CH), jnp.int32),
            pltpu.VMEM((GRP, CH), jnp.int32),
            pltpu.VMEM((GRP, CH), jnp.int32),
            pltpu.VMEM((GRP, CH), jnp.float32),
            pltpu.VMEM((GRP, CH), jnp.float32),
            pltpu.VMEM((CH, D), jnp.float32),
            pltpu.VMEM((CH, D), jnp.float32),
            pltpu.VMEM_SHARED((NPAD, D), jnp.float32),
            pltpu.SemaphoreType.DMA,
            pltpu.SemaphoreType.DMA,
            pltpu.SemaphoreType.DMA,
            pltpu.SemaphoreType.DMA,
            pltpu.SemaphoreType.DMA,
            pltpu.SemaphoreType.DMA,
        ],
        compiler_params=_SC_PARAMS,
    )
    def step(src_hbm, dst_hbm, w_hbm, hin_hbm, pout_hbm, htab_hbm,
             src0_v, src1_v, dst0_v, dst1_v, w0_v, w1_v, rows_a, rows_b,
             acc_sh, sem_a, sem_b, sem_i0, sem_i1, sem_wa, sem_wb):
        cid = lax.axis_index("c")
        sid = lax.axis_index("s")
        wid = cid * NS + sid
        base = sid * NODES_PER_TILE
        zeros = jnp.zeros((L,), jnp.float32)
        sbuf = (src0_v, src1_v)
        dbuf = (dst0_v, dst1_v)
        wbuf = (w0_v, w1_v)
        ibuf_sem = (sem_i0, sem_i1)
        htab = htab_hbm.at[cid]

        for b in range(NODES_PER_TILE // CH):
            sl = pl.ds(base + b * CH, CH)
            if first:
                pltpu.sync_copy(hin_hbm.at[sl], rows_a)
            else:
                pltpu.sync_copy(hin_hbm.at[0, sl], rows_a)
                pltpu.sync_copy(hin_hbm.at[1, sl], rows_b)
                def tanh_row(r, _):
                    for v in range(D // L):
                        cs = pl.ds(v * L, L)
                        s = rows_a[r, cs] + rows_b[r, cs]
                        e2 = jnp.exp(s * 2.0)
                        rows_a[r, cs] = 1.0 - 2.0 / (e2 + 1.0)
                    return 0
                lax.fori_loop(0, CH, tanh_row, 0)
            pltpu.sync_copy(rows_a, htab.at[sl])
        def z_rows(r, _):
            for v in range(D // L):
                rows_a[r, pl.ds(v * L, L)] = zeros
            return 0
        lax.fori_loop(0, CH, z_rows, 0)
        for b in range(NODES_PER_TILE // CH):
            pltpu.sync_copy(rows_a, acc_sh.at[pl.ds(base + b * CH, CH)])
        plsc.subcore_barrier()

        def issue_grp(g, b):
            gs = pl.ds(g * GRP, GRP)
            pltpu.async_copy(src_hbm.at[wid, gs], sbuf[b], ibuf_sem[b])
            pltpu.async_copy(dst_hbm.at[wid, gs], dbuf[b], ibuf_sem[b])
            pltpu.async_copy(w_hbm.at[wid, gs], wbuf[b], ibuf_sem[b])

        def wait_grp(g, b):
            gs = pl.ds(g * GRP, GRP)
            pltpu.make_async_copy(src_hbm.at[wid, gs], sbuf[b],
                                  ibuf_sem[b]).wait()
            pltpu.make_async_copy(dst_hbm.at[wid, gs], dbuf[b],
                                  ibuf_sem[b]).wait()
            pltpu.make_async_copy(w_hbm.at[wid, gs], wbuf[b],
                                  ibuf_sem[b]).wait()

        def scale(wv, rows, jj):
            jb = jnp.full((L,), jj, jnp.int32)
            def scale4(ei, _):
                for u in range(4):
                    e = ei * 4 + u
                    wb = plsc.load_gather(
                        wv, [jb, jnp.full((L,), e, jnp.int32)])
                    for v in range(D // L):
                        sl = pl.ds(v * L, L)
                        rows[e, sl] = rows[e, sl] * wb
                return 0
            lax.fori_loop(0, CH // 4, scale4, 0)

        issue_grp(0, 0)
        for g in range(NGRP):
            b = g % 2
            sv, dv, wv = sbuf[b], dbuf[b], wbuf[b]
            wait_grp(g, b)
            if g + 1 < NGRP:
                issue_grp(g + 1, 1 - b)
            pltpu.async_copy(htab.at[sv.at[0]], rows_a, sem_a)
            def pair_body(p, _):
                j0 = 2 * p
                @pl.when(j0 >= 1)
                def _():
                    pltpu.make_async_copy(rows_b, acc_sh.at[dv.at[j0 - 1]],
                                          sem_wb).wait()
                pltpu.async_copy(htab.at[sv.at[j0 + 1]], rows_b, sem_b)
                pltpu.make_async_copy(htab.at[sv.at[j0]], rows_a,
                                      sem_a).wait()
                scale(wv, rows_a, j0)
                pltpu.async_copy(rows_a, acc_sh.at[dv.at[j0]], sem_wa,
                                 add=True)
                pltpu.make_async_copy(htab.at[sv.at[j0 + 1]], rows_b,
                                      sem_b).wait()
                scale(wv, rows_b, j0 + 1)
                pltpu.async_copy(rows_b, acc_sh.at[dv.at[j0 + 1]], sem_wb,
                                 add=True)
                pltpu.make_async_copy(rows_a, acc_sh.at[dv.at[j0]],
                                      sem_wa).wait()
                @pl.when(p < GRP // 2 - 1)
                def _():
                    pltpu.async_copy(htab.at[sv.at[j0 + 2]], rows_a, sem_a)
                return 0
            lax.fori_loop(0, GRP // 2, pair_body, 0)
            pltpu.make_async_copy(rows_b, acc_sh.at[dv.at[GRP - 1]],
                                  sem_wb).wait()
        plsc.subcore_barrier()
        pltpu.sync_copy(acc_sh.at[pl.ds(base, NODES_PER_TILE)],
                        pout_hbm.at[cid, pl.ds(base, NODES_PER_TILE)])
    return step


_prop_first = _make_prop_step(True)
_prop_next = _make_prop_step(False)


def _tail_body(p0_ref, p1_ref, abs_ref, nf_ref, ns_ref, masks_ref, h_ref):
    h_ref[...] = jnp.tanh(p0_ref[...] + p1_ref[...])
    h = h_ref[...]
    ab = abs_ref[...]
    s2 = jnp.sum(h * h, axis=1)
    iota = jax.lax.broadcasted_iota(jnp.int32, (N,), 0)
    seeds = []
    for _ in range(K):
        i = jnp.argmax(s2)
        seeds.append(h_ref[pl.ds(i, 1), :])
        s2 = jnp.where(iota == i, -jnp.inf, s2)
    seeds = jnp.concatenate(seeds, axis=0)
    logits = jax.lax.dot_general(seeds, h, (((1,), (1,)), ((), ())),
                                 preferred_element_type=jnp.float32)
    logits = logits * (1.0 / jnp.sqrt(jnp.float32(D)))
    m = jnp.max(logits, axis=1, keepdims=True)
    e = jnp.exp(logits - m)
    s = jnp.sum(e, axis=1, keepdims=True)
    masks = e / s
    masks_ref[...] = masks
    nf_ref[...] = jax.lax.dot_general(masks, ab, (((1,), (0,)), ((), ())),
                                      preferred_element_type=jnp.float32)
    ns_ref[...] = jnp.max(masks, axis=1)


@jax.jit
def _tail(p0, p1, abstract):
    nf, ns, masks, _h = pl.pallas_call(
        _tail_body,
        out_shape=(
            jax.ShapeDtypeStruct((K, D), jnp.float32),
            jax.ShapeDtypeStruct((K,), jnp.float32),
            jax.ShapeDtypeStruct((K, N), jnp.float32),
            jax.ShapeDtypeStruct((N, D), jnp.float32),
        ),
    )(p0, p1, abstract)
    return nf, ns, masks


def kernel(x, edge_index, init_state, Wg, bg, Wk, bk, Wq, bq):
    src = edge_index[0]
    dst = edge_index[1]
    pad = EP - E
    src_p = jnp.concatenate([src, jnp.zeros((pad,), jnp.int32)])
    dst_p = jnp.concatenate([dst, jnp.full((pad,), DUMP, jnp.int32)])
    src3 = src_p.reshape(NW, NCHUNK, CH)
    dst3 = dst_p.reshape(NW, NCHUNK, CH)
    x_pad = jnp.concatenate([x, jnp.zeros((NPAD - N, D), jnp.float32)])
    h0_pad = jnp.concatenate([init_state,
                              jnp.zeros((NPAD - N, D), jnp.float32)])
    wcat = jnp.concatenate([Wg, Wk, Wq], axis=1)
    bcat = jnp.concatenate([bg, bk, bq])

    norm_full, inv_full, normx = _deg_stage(dst_p, x_pad)
    partials = _agg_sc(src3, dst3, normx)
    abstract, kn, qn = _encode(partials, x_pad, norm_full, inv_full, wcat,
                               bcat)
    cos3, m_part = _cos_sc(src3, dst3, kn, qn)
    w3 = _w_sc(dst3, cos3, m_part)

    pad2 = EPP - EP
    src_pp = jnp.concatenate([src_p, jnp.zeros((pad2,), jnp.int32)])
    dst_pp = jnp.concatenate([dst_p, jnp.full((pad2,), DUMP, jnp.int32)])
    w_pp = jnp.concatenate([w3.reshape(EP), jnp.zeros((pad2,), jnp.float32)])
    src4 = src_pp.reshape(NW, NCHP, CH)
    dst4 = dst_pp.reshape(NW, NCHP, CH)
    w4 = w_pp.reshape(NW, NCHP, CH)
    p, _ht = _prop_first(src4, dst4, w4, h0_pad)
    for _ in range(ITERS - 1):
        p, _ht = _prop_next(src4, dst4, w4, p)
    return _tail(p[0][:N], p[1][:N], abstract[:N])
